# baseline (speedup 1.0000x reference)
import jax
import jax.numpy as jnp
import numpy as np
from jax.experimental import pallas as pl
from jax.experimental.pallas import tpu as pltpu
from jax.experimental.pallas import tpu_sc as plsc

INIT_ALPHA = 0.0
LOSS_WEIGHT = 1.0
FOCAL_GAMMA = 0.0
PENALIZE_RATIO = 0.5
BLOCK_SIZE = (0.1, 1.0, 1.0)
MARGIN = 0.05


def _two_sum(a, b):
    s = a + b
    bb = s - a
    return s, (a - (s - bb)) + (b - bb)


def _split(a):
    c = jnp.float32(4097.0) * a
    h = c - (c - a)
    return h, a - h


def _two_prod(a, b):
    p = a * b
    ah, al = _split(a)
    bh, bl = _split(b)
    return p, ((ah * bh - p) + ah * bl + al * bh) + al * bl


def _ds_add(a, b):
    s, e = _two_sum(a[0], b[0])
    e = e + a[1] + b[1]
    hi = s + e
    return hi, e - (hi - s)


def _ds_mul(a, b):
    p, e = _two_prod(a[0], b[0])
    e = e + a[0] * b[1] + a[1] * b[0]
    hi = p + e
    return hi, e - (hi - p)


def _ds_neg(a):
    return -a[0], -a[1]


def _ds_const(x):
    h = np.float32(x)
    return jnp.float32(h), jnp.float32(np.float64(x) - np.float64(h))


def _select_masks(pred, tgt, block_size):
    np.random.seed(0)
    bounds = []
    for i in range(3):
        lo = jnp.min(pred[:, :, i], axis=1)
        hi = jnp.max(pred[:, :, i], axis=1)
        w = hi - lo
        z = jnp.zeros_like(lo)
        wm = _ds_mul((w, z), _ds_const(MARGIN))
        bounds.append([_ds_add((lo, z), wm), _ds_add((hi, z), _ds_neg(wm))])
    dim_size = [int(1.0 / s) for s in block_size]
    rsel = [np.random.randint(d) for d in dim_size]
    ranges = []
    for p, r in enumerate(rsel):
        span = _ds_add(bounds[p][1], _ds_neg(bounds[p][0]))
        bs = _ds_const(block_size[p])
        _min = _ds_add(_ds_mul(_ds_mul(span, _ds_const(float(r))), bs), bounds[p][0])
        _max = _ds_add(_min, _ds_mul(span, bs))
        ranges.append([_min, _max])

    def indicator(pts):
        ind = jnp.ones(pts.shape[:2], dtype=bool)
        for p in range(3):
            th0, tl0 = ranges[p][0]
            th1, tl1 = ranges[p][1]
            x = pts[:, :, p]
            gt = (x > th0[:, None]) | ((x == th0[:, None]) & (tl0[:, None] < 0))
            lt = (x < th1[:, None]) | ((x == th1[:, None]) & (tl1[:, None] > 0))
            ind = ind & gt & lt
        return ind

    return indicator(pred), indicator(tgt)


_N_SRC = 20000
_G16 = _N_SRC // 16


def _pack_sc_kernel(n_arr, src_hbm, mask_hbm, out_hbm, counts_hbm,
                    vals_v, mask_v, buf_v, cvec_v):
    wid = jax.lax.axis_index("s") * 2 + jax.lax.axis_index("c")
    arr = wid // 3
    coord = wid % 3

    @pl.when(wid < n_arr * 3)
    def _():
        pltpu.sync_copy(src_hbm.at[arr, coord], vals_v)
        pltpu.sync_copy(mask_hbm.at[arr], mask_v)

        def body(i, base):
            off = i * 16
            v = vals_v[pl.ds(off, 16)]
            m = mask_v[pl.ds(off, 16)] != 0
            plsc.store_compressed(buf_v.at[pl.ds(base, 16)], v, mask=m)
            return base + jnp.sum(m.astype(jnp.int32))

        cnt = jax.lax.fori_loop(0, _G16, body, jnp.int32(0), unroll=False)

        @pl.when(coord == 0)
        def _():
            cvec_v[...] = jnp.full((16,), cnt, jnp.int32)
            pltpu.sync_copy(cvec_v, counts_hbm.at[arr])

        n_big = cnt // 512

        def big_body(i, c):
            pltpu.sync_copy(buf_v.at[pl.ds(i * 512, 512)],
                            out_hbm.at[arr, coord, pl.ds(i * 512, 512)])
            return c

        jax.lax.fori_loop(0, n_big, big_body, 0, unroll=False)
        off1 = n_big * 512
        n_mid = (cnt - off1) // 64

        def mid_body(i, c):
            pltpu.sync_copy(buf_v.at[pl.ds(off1 + i * 64, 64)],
                            out_hbm.at[arr, coord, pl.ds(off1 + i * 64, 64)])
            return c

        jax.lax.fori_loop(0, n_mid, mid_body, 0, unroll=False)
        off2 = off1 + n_mid * 64
        n_sm = (cnt - off2 + 7) // 8

        def sm_body(i, c):
            pltpu.sync_copy(buf_v.at[pl.ds(off2 + i * 8, 8)],
                            out_hbm.at[arr, coord, pl.ds(off2 + i * 8, 8)])
            return c

        jax.lax.fori_loop(0, n_sm, sm_body, 0, unroll=False)


def _pack_sc(src, masks):
    import dataclasses
    import functools
    n_arr = src.shape[0]
    cp = pltpu.CompilerParams()
    if "needs_layout_passes" in pltpu.CompilerParams.__dataclass_fields__:
        cp = dataclasses.replace(cp, needs_layout_passes=False)
    mesh = plsc.VectorSubcoreMesh(core_axis_name="c", subcore_axis_name="s",
                                  num_cores=2, num_subcores=16)
    f = pl.kernel(
        functools.partial(_pack_sc_kernel, n_arr),
        out_type=[
            jax.ShapeDtypeStruct((n_arr, 3, N_PAD), jnp.float32),
            jax.ShapeDtypeStruct((n_arr, 16), jnp.int32),
        ],
        mesh=mesh,
        scratch_types=[
            pltpu.VMEM((_N_SRC,), jnp.float32),
            pltpu.VMEM((_N_SRC,), jnp.int32),
            pltpu.VMEM((_N_SRC + 16,), jnp.float32),
            pltpu.VMEM((16,), jnp.int32),
        ],
        compiler_params=cp,
    )
    return f(src, masks)


N_PAD = 20480
TT = 128
PT = 128
ROWS = N_PAD // PT
_INF_BITS = np.int32(0x7F800000)


def _chamfer_kernel(nb, counts_ref, pred_ref, tgt_ref, out_ref, diff_ref,
                    bits_ref, tsan_ref):
    b = pl.program_id(0)
    cnt_p = counts_ref[b, 0]
    cnt_t = counts_ref[b + nb, 0]
    p_tiles = (cnt_p + PT - 1) // PT
    t_tiles = (cnt_t + TT - 1) // TT

    diff_ref[...] = jnp.full((ROWS, PT), jnp.inf, jnp.float32)

    lane_iota = jax.lax.broadcasted_iota(jnp.int32, (TT,), 0)

    def san_body(r, _):
        valid = (r * TT + lane_iota) < cnt_t
        tsan_ref[0, r, :] = jnp.where(valid, tgt_ref[0, 0, pl.ds(r * TT, TT)],
                                      jnp.float32(1e9))
        tsan_ref[1, r, :] = jnp.where(valid, tgt_ref[0, 1, pl.ds(r * TT, TT)],
                                      jnp.float32(1e9))
        tsan_ref[2, r, :] = jnp.where(valid, tgt_ref[0, 2, pl.ds(r * TT, TT)],
                                      jnp.float32(1e9))
        return 0

    jax.lax.fori_loop(0, t_tiles, san_body, 0, unroll=False)

    def p_body(pi, _):
        poff = pi * PT
        px = jnp.broadcast_to(
            pred_ref[0, 0, pl.ds(poff, PT)].reshape(PT, 1), (PT, TT))
        py = jnp.broadcast_to(
            pred_ref[0, 1, pl.ds(poff, PT)].reshape(PT, 1), (PT, TT))
        pz = jnp.broadcast_to(
            pred_ref[0, 2, pl.ds(poff, PT)].reshape(PT, 1), (PT, TT))

        def t_step(ti, acc):
            tx = tsan_ref[0, ti, :].reshape(1, TT)
            ty = tsan_ref[1, ti, :].reshape(1, TT)
            tz = tsan_ref[2, ti, :].reshape(1, TT)
            dx = px - tx
            d = dx * dx
            dy = py - ty
            d = d + dy * dy
            dz = pz - tz
            d = d + dz * dz
            return jnp.minimum(acc, d)

        def t_body2(i, acc):
            return t_step(2 * i + 1, t_step(2 * i, acc))

        acc = jax.lax.fori_loop(
            0, t_tiles // 2, t_body2,
            jnp.full((PT, TT), jnp.inf, jnp.float32), unroll=False)
        acc = jax.lax.cond(t_tiles % 2 == 1,
                           lambda a: t_step(t_tiles - 1, a),
                           lambda a: a, acc)
        diff_ref[pi, :] = jnp.min(acc, axis=1)
        return 0

    jax.lax.fori_loop(0, p_tiles, p_body, 0, unroll=False)

    gidx = (jax.lax.broadcasted_iota(jnp.int32, (ROWS, PT), 0) * PT
            + jax.lax.broadcasted_iota(jnp.int32, (ROWS, PT), 1))
    diff = jnp.where(gidx < cnt_p, diff_ref[...], jnp.inf)
    diff_ref[...] = diff
    bits_ref[...] = jax.lax.bitcast_convert_type(diff, jnp.int32)
    bits = bits_ref[...]

    k = 1 + cnt_p // 2

    def bs_body(_, carry):
        lo, hi = carry
        mid = lo + (hi - lo) // 2
        c = jnp.sum((bits <= mid).astype(jnp.int32))
        ge = c >= k
        new_lo = jnp.where(ge, lo, mid + 1)
        new_hi = jnp.where(ge, mid, hi)
        return new_lo, new_hi

    m_bits, _ = jax.lax.fori_loop(
        0, 32, bs_body, (jnp.int32(0), jnp.int32(_INF_BITS)))

    keep = bits < m_bits
    cnt = jnp.sum(keep.astype(jnp.float32))
    sum_sq = jnp.sum(jnp.where(keep, diff * diff, jnp.float32(0.0)))
    loss_b = sum_sq / (cnt + 1e-12)
    out_ref[0, 0, :] = jnp.full((128,), loss_b, jnp.float32)


def _chamfer_losses(counts16, packed):
    import functools
    B = packed.shape[0] // 2
    grid_spec = pltpu.PrefetchScalarGridSpec(
        num_scalar_prefetch=1,
        grid=(B,),
        in_specs=[
            pl.BlockSpec((1, 3, N_PAD), lambda b, c: (b, 0, 0)),
            pl.BlockSpec((1, 3, N_PAD), lambda b, c: (b + B, 0, 0)),
        ],
        out_specs=pl.BlockSpec((1, 1, 128), lambda b, c: (b, 0, 0)),
        scratch_shapes=[
            pltpu.VMEM((ROWS, PT), jnp.float32),
            pltpu.VMEM((ROWS, PT), jnp.int32),
            pltpu.VMEM((3, ROWS, TT), jnp.float32),
        ],
    )
    out = pl.pallas_call(
        functools.partial(_chamfer_kernel, B),
        grid_spec=grid_spec,
        out_shape=jax.ShapeDtypeStruct((B, 1, 128), jnp.float32),
    )(counts16, packed, packed)
    return out[:, 0, 0]


def kernel(prediction_tensor, target_tensor, alpha):
    ind_pred, ind_tgt = _select_masks(prediction_tensor, target_tensor, BLOCK_SIZE)
    B, N, _ = prediction_tensor.shape
    T = target_tensor.shape[1]
    predT = prediction_tensor.transpose(0, 2, 1)
    tgtT = target_tensor.transpose(0, 2, 1)

    cnt_p_raw = jnp.sum(ind_pred, axis=1)
    cnt_t_raw = jnp.sum(ind_tgt, axis=1)
    maskp_eff = ind_pred & (cnt_p_raw >= 500)[:, None]
    maskt_eff = ind_tgt | (cnt_t_raw < 500)[:, None]
    h = B // 2
    lbs = []
    for lo, hi in ((0, h), (h, B)):
        src = jnp.concatenate([predT[lo:hi], tgtT[lo:hi]], axis=0)
        masks = jnp.concatenate(
            [maskp_eff[lo:hi], maskt_eff[lo:hi]], axis=0).astype(jnp.int32)
        packed, counts16 = _pack_sc(src, masks)
        lbs.append(_chamfer_losses(counts16, packed))
    lb = jnp.concatenate(lbs)

    loss = jnp.float32(0.0)
    for b in range(B):
        loss = loss + lb[b]
    loss = loss / B
    focal_weight = (jnp.exp(-alpha) * loss) ** FOCAL_GAMMA
    focal_weight = focal_weight / (jnp.sum(focal_weight) + 1e-12)
    loss = focal_weight * (jnp.exp(-alpha) * loss)
    loss = jnp.sum(loss) + alpha
    return LOSS_WEIGHT * loss


# speedup vs baseline: 1.0450x; 1.0450x over previous
import jax
import jax.numpy as jnp
import numpy as np
from jax.experimental import pallas as pl
from jax.experimental.pallas import tpu as pltpu
from jax.experimental.pallas import tpu_sc as plsc

INIT_ALPHA = 0.0
LOSS_WEIGHT = 1.0
FOCAL_GAMMA = 0.0
PENALIZE_RATIO = 0.5
BLOCK_SIZE = (0.1, 1.0, 1.0)
MARGIN = 0.05


def _two_sum(a, b):
    s = a + b
    bb = s - a
    return s, (a - (s - bb)) + (b - bb)


def _split(a):
    c = jnp.float32(4097.0) * a
    h = c - (c - a)
    return h, a - h


def _two_prod(a, b):
    p = a * b
    ah, al = _split(a)
    bh, bl = _split(b)
    return p, ((ah * bh - p) + ah * bl + al * bh) + al * bl


def _ds_add(a, b):
    s, e = _two_sum(a[0], b[0])
    e = e + a[1] + b[1]
    hi = s + e
    return hi, e - (hi - s)


def _ds_mul(a, b):
    p, e = _two_prod(a[0], b[0])
    e = e + a[0] * b[1] + a[1] * b[0]
    hi = p + e
    return hi, e - (hi - p)


def _ds_neg(a):
    return -a[0], -a[1]


def _ds_const(x):
    h = np.float32(x)
    return jnp.float32(h), jnp.float32(np.float64(x) - np.float64(h))


def _select_masks(pred, tgt, block_size):
    np.random.seed(0)
    bounds = []
    for i in range(3):
        lo = jnp.min(pred[:, :, i], axis=1)
        hi = jnp.max(pred[:, :, i], axis=1)
        w = hi - lo
        z = jnp.zeros_like(lo)
        wm = _ds_mul((w, z), _ds_const(MARGIN))
        bounds.append([_ds_add((lo, z), wm), _ds_add((hi, z), _ds_neg(wm))])
    dim_size = [int(1.0 / s) for s in block_size]
    rsel = [np.random.randint(d) for d in dim_size]
    ranges = []
    for p, r in enumerate(rsel):
        span = _ds_add(bounds[p][1], _ds_neg(bounds[p][0]))
        bs = _ds_const(block_size[p])
        _min = _ds_add(_ds_mul(_ds_mul(span, _ds_const(float(r))), bs), bounds[p][0])
        _max = _ds_add(_min, _ds_mul(span, bs))
        ranges.append([_min, _max])

    def indicator(pts):
        ind = jnp.ones(pts.shape[:2], dtype=bool)
        for p in range(3):
            th0, tl0 = ranges[p][0]
            th1, tl1 = ranges[p][1]
            x = pts[:, :, p]
            gt = (x > th0[:, None]) | ((x == th0[:, None]) & (tl0[:, None] < 0))
            lt = (x < th1[:, None]) | ((x == th1[:, None]) & (tl1[:, None] > 0))
            ind = ind & gt & lt
        return ind

    return indicator(pred), indicator(tgt)


_N_SRC = 20000
_G16 = _N_SRC // 16


def _pack_sc_kernel(n_arr, src_hbm, mask_hbm, out_hbm, counts_hbm,
                    vals_v, mask_v, buf_v, cvec_v):
    wid = jax.lax.axis_index("s") * 2 + jax.lax.axis_index("c")
    arr = wid // 3
    coord = wid % 3

    @pl.when(wid < n_arr * 3)
    def _():
        pltpu.sync_copy(src_hbm.at[arr, coord], vals_v)
        pltpu.sync_copy(mask_hbm.at[arr], mask_v)

        def body(i, base):
            off = i * 16
            v = vals_v[pl.ds(off, 16)]
            m = mask_v[pl.ds(off, 16)] != 0
            plsc.store_compressed(buf_v.at[pl.ds(base, 16)], v, mask=m)
            return base + jnp.sum(m.astype(jnp.int32))

        cnt = jax.lax.fori_loop(0, _G16, body, jnp.int32(0), unroll=False)

        @pl.when(coord == 0)
        def _():
            cvec_v[...] = jnp.full((16,), cnt, jnp.int32)
            pltpu.sync_copy(cvec_v, counts_hbm.at[arr])

        n_big = cnt // 512

        def big_body(i, c):
            pltpu.sync_copy(buf_v.at[pl.ds(i * 512, 512)],
                            out_hbm.at[arr, coord, pl.ds(i * 512, 512)])
            return c

        jax.lax.fori_loop(0, n_big, big_body, 0, unroll=False)
        off1 = n_big * 512
        n_mid = (cnt - off1) // 64

        def mid_body(i, c):
            pltpu.sync_copy(buf_v.at[pl.ds(off1 + i * 64, 64)],
                            out_hbm.at[arr, coord, pl.ds(off1 + i * 64, 64)])
            return c

        jax.lax.fori_loop(0, n_mid, mid_body, 0, unroll=False)
        off2 = off1 + n_mid * 64
        n_sm = (cnt - off2 + 7) // 8

        def sm_body(i, c):
            pltpu.sync_copy(buf_v.at[pl.ds(off2 + i * 8, 8)],
                            out_hbm.at[arr, coord, pl.ds(off2 + i * 8, 8)])
            return c

        jax.lax.fori_loop(0, n_sm, sm_body, 0, unroll=False)


def _pack_sc(src, masks):
    import dataclasses
    import functools
    n_arr = src.shape[0]
    cp = pltpu.CompilerParams()
    if "needs_layout_passes" in pltpu.CompilerParams.__dataclass_fields__:
        cp = dataclasses.replace(cp, needs_layout_passes=False)
    mesh = plsc.VectorSubcoreMesh(core_axis_name="c", subcore_axis_name="s",
                                  num_cores=2, num_subcores=16)
    f = pl.kernel(
        functools.partial(_pack_sc_kernel, n_arr),
        out_type=[
            jax.ShapeDtypeStruct((n_arr, 3, N_PAD), jnp.float32),
            jax.ShapeDtypeStruct((n_arr, 16), jnp.int32),
        ],
        mesh=mesh,
        scratch_types=[
            pltpu.VMEM((_N_SRC,), jnp.float32),
            pltpu.VMEM((_N_SRC,), jnp.int32),
            pltpu.VMEM((_N_SRC + 16,), jnp.float32),
            pltpu.VMEM((16,), jnp.int32),
        ],
        compiler_params=cp,
    )
    return f(src, masks)


N_PAD = 20480
TT = 128
PT = 128
ROWS = N_PAD // PT
_INF_BITS = np.int32(0x7F800000)


def _chamfer_kernel(nb, counts_ref, pred_ref, tgt_ref, out_ref, diff_ref,
                    bits_ref, tsan_ref):
    b = pl.program_id(0)
    cnt_p = counts_ref[b, 0]
    cnt_t = counts_ref[b + nb, 0]
    p_tiles = (cnt_p + PT - 1) // PT
    t_tiles = (cnt_t + TT - 1) // TT

    diff_ref[...] = jnp.full((ROWS, PT), jnp.inf, jnp.float32)

    lane_iota = jax.lax.broadcasted_iota(jnp.int32, (TT,), 0)

    def san_body(r, _):
        valid = (r * TT + lane_iota) < cnt_t
        tsan_ref[0, r, :] = jnp.where(valid, tgt_ref[0, 0, pl.ds(r * TT, TT)],
                                      jnp.float32(1e9))
        tsan_ref[1, r, :] = jnp.where(valid, tgt_ref[0, 1, pl.ds(r * TT, TT)],
                                      jnp.float32(1e9))
        tsan_ref[2, r, :] = jnp.where(valid, tgt_ref[0, 2, pl.ds(r * TT, TT)],
                                      jnp.float32(1e9))
        return 0

    jax.lax.fori_loop(0, t_tiles, san_body, 0, unroll=False)

    def p_body(pi, _):
        poff = pi * PT
        px = jnp.broadcast_to(
            pred_ref[0, 0, pl.ds(poff, PT)].reshape(PT, 1), (PT, TT))
        py = jnp.broadcast_to(
            pred_ref[0, 1, pl.ds(poff, PT)].reshape(PT, 1), (PT, TT))
        pz = jnp.broadcast_to(
            pred_ref[0, 2, pl.ds(poff, PT)].reshape(PT, 1), (PT, TT))

        def t_step(ti, acc):
            tx = tsan_ref[0, ti, :].reshape(1, TT)
            ty = tsan_ref[1, ti, :].reshape(1, TT)
            tz = tsan_ref[2, ti, :].reshape(1, TT)
            dx = px - tx
            d = dx * dx
            dy = py - ty
            d = d + dy * dy
            dz = pz - tz
            d = d + dz * dz
            return jnp.minimum(acc, d)

        def t_body2(i, acc):
            return t_step(2 * i + 1, t_step(2 * i, acc))

        acc = jax.lax.fori_loop(
            0, t_tiles // 2, t_body2,
            jnp.full((PT, TT), jnp.inf, jnp.float32), unroll=False)
        acc = jax.lax.cond(t_tiles % 2 == 1,
                           lambda a: t_step(t_tiles - 1, a),
                           lambda a: a, acc)
        diff_ref[pi, :] = jnp.min(acc, axis=1)
        return 0

    jax.lax.fori_loop(0, p_tiles, p_body, 0, unroll=False)

    gidx = (jax.lax.broadcasted_iota(jnp.int32, (ROWS, PT), 0) * PT
            + jax.lax.broadcasted_iota(jnp.int32, (ROWS, PT), 1))
    diff = jnp.where(gidx < cnt_p, diff_ref[...], jnp.inf)
    diff_ref[...] = diff
    bits_ref[...] = jax.lax.bitcast_convert_type(diff, jnp.int32)
    bits = bits_ref[...]

    k = 1 + cnt_p // 2

    def bs_body(_, carry):
        lo, hi = carry
        mid = lo + (hi - lo) // 2
        c = jnp.sum((bits <= mid).astype(jnp.int32))
        ge = c >= k
        new_lo = jnp.where(ge, lo, mid + 1)
        new_hi = jnp.where(ge, mid, hi)
        return new_lo, new_hi

    m_bits, _ = jax.lax.fori_loop(
        0, 32, bs_body, (jnp.int32(0), jnp.int32(_INF_BITS)))

    keep = bits < m_bits
    cnt = jnp.sum(keep.astype(jnp.float32))
    sum_sq = jnp.sum(jnp.where(keep, diff * diff, jnp.float32(0.0)))
    loss_b = sum_sq / (cnt + 1e-12)
    out_ref[0, 0, :] = jnp.full((128,), loss_b, jnp.float32)


def _chamfer_losses(counts16, packed):
    import functools
    B = packed.shape[0] // 2
    grid_spec = pltpu.PrefetchScalarGridSpec(
        num_scalar_prefetch=1,
        grid=(B,),
        in_specs=[
            pl.BlockSpec((1, 3, N_PAD), lambda b, c: (b, 0, 0)),
            pl.BlockSpec((1, 3, N_PAD), lambda b, c: (b + B, 0, 0)),
        ],
        out_specs=pl.BlockSpec((1, 1, 128), lambda b, c: (b, 0, 0)),
        scratch_shapes=[
            pltpu.VMEM((ROWS, PT), jnp.float32),
            pltpu.VMEM((ROWS, PT), jnp.int32),
            pltpu.VMEM((3, ROWS, TT), jnp.float32),
        ],
    )
    out = pl.pallas_call(
        functools.partial(_chamfer_kernel, B),
        grid_spec=grid_spec,
        out_shape=jax.ShapeDtypeStruct((B, 1, 128), jnp.float32),
    )(counts16, packed, packed)
    return out[:, 0, 0]


def kernel(prediction_tensor, target_tensor, alpha):
    ind_pred, ind_tgt = _select_masks(prediction_tensor, target_tensor, BLOCK_SIZE)
    B, N, _ = prediction_tensor.shape
    T = target_tensor.shape[1]
    predT = prediction_tensor.transpose(0, 2, 1)
    tgtT = target_tensor.transpose(0, 2, 1)

    cnt_p_raw = jnp.sum(ind_pred, axis=1)
    cnt_t_raw = jnp.sum(ind_tgt, axis=1)
    maskp_eff = ind_pred & (cnt_p_raw >= 500)[:, None]
    maskt_eff = ind_tgt | (cnt_t_raw < 500)[:, None]
    src = jnp.concatenate([predT, tgtT], axis=0)
    masks = jnp.concatenate([maskp_eff, maskt_eff], axis=0).astype(jnp.int32)

    packed, counts16 = _pack_sc(src, masks)
    lb = _chamfer_losses(counts16, packed)

    loss = jnp.float32(0.0)
    for b in range(B):
        loss = loss + lb[b]
    loss = loss / B
    focal_weight = (jnp.exp(-alpha) * loss) ** FOCAL_GAMMA
    focal_weight = focal_weight / (jnp.sum(focal_weight) + 1e-12)
    loss = focal_weight * (jnp.exp(-alpha) * loss)
    loss = jnp.sum(loss) + alpha
    return LOSS_WEIGHT * loss
